# Initial kernel scaffold; baseline (speedup 1.0000x reference)
import jax
import jax.numpy as jnp
from jax.experimental import pallas as pl


def kernel(r, edge_index, phi_density, phi_pair, emb_weights):
    raise NotImplementedError("write your pallas kernel here")


# speedup vs baseline: 9.4099x; 9.4099x over previous
import functools

import jax
import jax.numpy as jnp
import numpy as np
from jax import lax
from jax.experimental import pallas as pl
from jax.experimental.pallas import tpu as pltpu
from jax.experimental.pallas import tpu_sc as plsc

NBASIS = 128
CUTOFF = 6.0
N_NODES = 10000
N_EDGES = 320000

_DELTA = CUTOFF / (NBASIS - 1)
_GAMMA = 1.0 / _DELTA
_PI = float(np.pi)

_SUB_TOTAL = N_EDGES // 128
_BLK_SUB = 25
_GRID = _SUB_TOTAL // _BLK_SUB


def _edge_body(sp_ref, pp_ref, rx_ref, ry_ref, rz_ref,
               d_ref, c1_ref, c0_ref, ps_ref):
    rx = rx_ref[...]
    ry = ry_ref[...]
    rz = rz_ref[...]
    l2 = rx * rx + ry * ry + rz * rz
    inv_l = lax.rsqrt(l2)
    bl = l2 * inv_l
    th = (_PI / CUTOFF) * bl
    fc = 0.5 + 0.5 * jnp.cos(th)
    fcp = (-_PI / (2.0 * CUTOFF)) * jnp.sin(th)

    zero = jnp.zeros_like(bl)

    def basis_step(k, carry):
        sa, sac, sb, sbc = carry
        ck = k.astype(jnp.float32) * _DELTA
        t = bl - ck
        e = jnp.exp((-_GAMMA) * (t * t))
        spk = sp_ref[k]
        ppk = pp_ref[k]
        sa = sa + spk * e
        sac = sac + (spk * ck) * e
        sb = sb + ppk * e
        sbc = sbc + (ppk * ck) * e
        return sa, sac, sb, sbc

    sa, sac, sb, sbc = lax.fori_loop(0, NBASIS, basis_step,
                                     (zero, zero, zero, zero))

    ap = (-2.0 * _GAMMA) * (bl * sa - sac)
    bp = (-2.0 * _GAMMA) * (bl * sb - sbc)
    d = 0.5 * sa * fc
    c1 = 0.5 * (ap * fc + sa * fcp) * inv_l
    p = 0.5 * sb * fc * inv_l
    c0 = ((bp * fc + sb * fcp) * 0.5 * inv_l - p * inv_l) * inv_l

    d_ref[...] = d
    c1_ref[...] = c1
    c0_ref[...] = c0

    @pl.when(pl.program_id(0) == 0)
    def _():
        ps_ref[0, 0] = 0.0

    ps_ref[0, 0] += jnp.sum(p)


def _edge_stage(sp, pp, rx2, ry2, rz2):
    blk = pl.BlockSpec((1, _BLK_SUB, 128), lambda i: (i, 0, 0))
    return pl.pallas_call(
        _edge_body,
        grid=(_GRID,),
        in_specs=[
            pl.BlockSpec(memory_space=pltpu.SMEM),
            pl.BlockSpec(memory_space=pltpu.SMEM),
            blk, blk, blk,
        ],
        out_specs=[
            blk, blk, blk,
            pl.BlockSpec((1, 1), lambda i: (0, 0), memory_space=pltpu.SMEM),
        ],
        out_shape=[
            jax.ShapeDtypeStruct((_GRID, _BLK_SUB, 128), jnp.float32),
            jax.ShapeDtypeStruct((_GRID, _BLK_SUB, 128), jnp.float32),
            jax.ShapeDtypeStruct((_GRID, _BLK_SUB, 128), jnp.float32),
            jax.ShapeDtypeStruct((1, 1), jnp.float32),
        ],
    )(sp, pp, rx2, ry2, rz2)


_NT = 16
_EPAD = 327680
_EPW = _EPAD // _NT
_NSUB = 4
_CH = _EPW // _NSUB
_NPAD = 10240
_NPT = _NPAD // _NT


def _rsqrt_newton(x):
    m = x
    sc = jnp.full((16,), 1.0, jnp.float32)
    for _ in range(16):
        big = m >= 4.0
        m = jnp.where(big, m * 0.25, m)
        sc = jnp.where(big, sc * 0.5, sc)
    for _ in range(32):
        small = m < 1.0
        m = jnp.where(small, m * 4.0, m)
        sc = jnp.where(small, sc * 2.0, sc)
    y = 1.30880086 + m * (-0.39516662 + m * 0.04939814)
    for _ in range(3):
        y = y * (1.5 - (0.5 * m) * y * y)
    return y * sc


def _sc_body(d_hbm, c1_hbm, c0_hbm, rx_hbm, ry_hbm, rz_hbm,
             src_hbm, dst_hbm, coef_hbm, zn_hbm, zf_hbm,
             fout_hbm, esum_hbm,
             rho_sh, fp_sh, fpos_sh, fneg_sh, es_sh,
             idxd, idxs, v0, v1, vx, vy, vz, dxyz,
             fp_loc, rho_loc, fp_part, coef_v, va, vb, es_loc):
    wid = lax.axis_index("s")
    ebase = wid * _EPW
    nbase = wid * _NPT

    pltpu.sync_copy(zn_hbm, rho_loc)
    pltpu.sync_copy(rho_loc, rho_sh.at[pl.ds(nbase, _NPT)])
    pltpu.sync_copy(zf_hbm, va)
    pltpu.sync_copy(va, fpos_sh.at[pl.ds(nbase, _NPT)])
    pltpu.sync_copy(va, fneg_sh.at[pl.ds(nbase, _NPT)])
    for k in range(_CH // _NPT):
        pltpu.sync_copy(zf_hbm, dxyz.at[pl.ds(k * _NPT, _NPT)])
    pltpu.sync_copy(coef_hbm, coef_v)
    plsc.subcore_barrier()

    for s in range(_NSUB):
        off = ebase + s * _CH
        pltpu.sync_copy(d_hbm.at[pl.ds(off, _CH)], v0)
        pltpu.sync_copy(dst_hbm.at[pl.ds(off, _CH)], idxd)
        pltpu.sync_copy(v0, rho_sh.at[idxd], add=True)
    plsc.subcore_barrier()

    pltpu.sync_copy(rho_sh.at[pl.ds(nbase, _NPT)], rho_loc)
    cf = [coef_v[j] for j in range(5)]
    gf = [coef_v[5 + j] for j in range(5)]

    def node_step(i, facc):
        o = i * 16
        x = rho_loc[pl.ds(o, 16)]
        y = _rsqrt_newton(x)
        s = x * y
        fval = cf[0] * s + x * (cf[1] + x * (cf[2] + x * (cf[3] + x * cf[4])))
        fp = gf[0] * y + gf[1] + x * (gf[2] + x * (gf[3] + x * gf[4]))
        fp_part[pl.ds(o, 16)] = fp
        return facc + fval

    facc = lax.fori_loop(0, _NPT // 16, node_step, jnp.zeros((16,), jnp.float32))
    pltpu.sync_copy(fp_part, fp_sh.at[pl.ds(nbase, _NPT)])
    es_loc[0] = facc
    pltpu.sync_copy(es_loc.at[0], es_sh.at[wid])
    plsc.subcore_barrier()

    pltpu.sync_copy(fp_sh, fp_loc)
    iota = lax.iota(jnp.int32, 16)
    col0 = jnp.zeros((16,), jnp.int32)
    col1 = jnp.full((16,), 1, jnp.int32)
    col2 = jnp.full((16,), 2, jnp.int32)
    for s in range(_NSUB):
        off = ebase + s * _CH
        pltpu.sync_copy(c1_hbm.at[pl.ds(off, _CH)], v0)
        pltpu.sync_copy(c0_hbm.at[pl.ds(off, _CH)], v1)
        pltpu.sync_copy(dst_hbm.at[pl.ds(off, _CH)], idxd)
        pltpu.sync_copy(src_hbm.at[pl.ds(off, _CH)], idxs)
        pltpu.sync_copy(rx_hbm.at[pl.ds(off, _CH)], vx)
        pltpu.sync_copy(ry_hbm.at[pl.ds(off, _CH)], vy)
        pltpu.sync_copy(rz_hbm.at[pl.ds(off, _CH)], vz)

        def edge_step(j, _):
            o = j * 16
            dv = idxd[pl.ds(o, 16)]
            fpd = plsc.load_gather(fp_loc, ([dv]))
            g = fpd * v0[pl.ds(o, 16)] + v1[pl.ds(o, 16)]
            rows = iota + o
            plsc.store_scatter(dxyz, ([rows, col0]),
                               g * vx[pl.ds(o, 16)])
            plsc.store_scatter(dxyz, ([rows, col1]),
                               g * vy[pl.ds(o, 16)])
            plsc.store_scatter(dxyz, ([rows, col2]),
                               g * vz[pl.ds(o, 16)])
            return 0

        lax.fori_loop(0, _CH // 16, edge_step, 0)
        pltpu.sync_copy(dxyz, fpos_sh.at[idxs], add=True)
        pltpu.sync_copy(dxyz, fneg_sh.at[idxd], add=True)
    plsc.subcore_barrier()

    pltpu.sync_copy(fpos_sh.at[pl.ds(nbase, _NPT)], va)
    pltpu.sync_copy(fneg_sh.at[pl.ds(nbase, _NPT)], vb)

    def sub_step(m, _):
        c = m // (_NPT // 16)
        rows = iota + 16 * (m % (_NPT // 16))
        colv = jnp.full((16,), 1, jnp.int32) * c
        a = plsc.load_gather(va, ([rows, colv]))
        b = plsc.load_gather(vb, ([rows, colv]))
        plsc.store_scatter(va, ([rows, colv]), a - b)
        return 0

    lax.fori_loop(0, 3 * (_NPT // 16), sub_step, 0)
    pltpu.sync_copy(va, fout_hbm.at[pl.ds(nbase, _NPT)])

    @pl.when(wid == 0)
    def _():
        pltpu.sync_copy(es_sh, es_loc)
        acc = es_loc[0]
        for t in range(1, _NT):
            acc = acc + es_loc[t]
        tot = lax.broadcast(jnp.sum(acc, axis=0), (16,))
        es_loc[0] = tot
        pltpu.sync_copy(es_loc.at[0], esum_hbm)


def _sc_stage(d_e, c1_e, c0_e, rx, ry, rz, src, dst, coefs, zn, zf):
    mesh = plsc.VectorSubcoreMesh(core_axis_name="c", subcore_axis_name="s",
                                  num_cores=1, num_subcores=_NT)
    f32 = jnp.float32
    run = pl.kernel(
        _sc_body,
        out_type=[
            jax.ShapeDtypeStruct((_NPAD, 8), f32),
            jax.ShapeDtypeStruct((16,), f32),
        ],
        mesh=mesh,
        compiler_params=pltpu.CompilerParams(needs_layout_passes=False,
                                             use_tc_tiling_on_sc=False),
        scratch_types=[
            pltpu.VMEM_SHARED((_NPAD,), f32),
            pltpu.VMEM_SHARED((_NPAD,), f32),
            pltpu.VMEM_SHARED((_NPAD, 8), f32),
            pltpu.VMEM_SHARED((_NPAD, 8), f32),
            pltpu.VMEM_SHARED((_NT, 16), f32),
            pltpu.VMEM((_CH,), jnp.int32),
            pltpu.VMEM((_CH,), jnp.int32),
            pltpu.VMEM((_CH,), f32),
            pltpu.VMEM((_CH,), f32),
            pltpu.VMEM((_CH,), f32),
            pltpu.VMEM((_CH,), f32),
            pltpu.VMEM((_CH,), f32),
            pltpu.VMEM((_CH, 8), f32),
            pltpu.VMEM((_NPAD,), f32),
            pltpu.VMEM((_NPT,), f32),
            pltpu.VMEM((_NPT,), f32),
            pltpu.VMEM((10, 16), f32),
            pltpu.VMEM((_NPT, 8), f32),
            pltpu.VMEM((_NPT, 8), f32),
            pltpu.VMEM((_NT, 16), f32),
        ],
    )
    return run(d_e, c1_e, c0_e, rx, ry, rz, src, dst, coefs, zn, zf)


_POWERS = np.concatenate([[0.5], 1.0 + np.arange(4)]).astype(np.float32)
_SF = np.concatenate(
    [[2.0], 1.0 / 10.0 ** np.cumsum(np.log10(1.0 + np.arange(4)))]
).astype(np.float32)


@jax.jit
def kernel(r, edge_index, phi_density, phi_pair, emb_weights):
    sp = jax.nn.softplus(phi_density)
    pp = phi_pair

    rx2 = r[:, 0].reshape(_GRID, _BLK_SUB, 128)
    ry2 = r[:, 1].reshape(_GRID, _BLK_SUB, 128)
    rz2 = r[:, 2].reshape(_GRID, _BLK_SUB, 128)

    d2, c12, c02, psum = _edge_stage(sp, pp, rx2, ry2, rz2)

    cf = emb_weights * jnp.asarray(_SF)
    gfc = cf * jnp.asarray(_POWERS)
    coefs = jnp.broadcast_to(
        jnp.concatenate([cf, gfc])[:, None], (10, 16)).astype(jnp.float32)

    src = edge_index[0].astype(jnp.int32)
    dst = edge_index[1].astype(jnp.int32)
    zn = jnp.zeros((_NPT,), jnp.float32)
    zf = jnp.zeros((_NPT, 8), jnp.float32)

    npad = _EPAD - N_EDGES
    padf = lambda a: jnp.pad(a, (0, npad))

    fout, esum = _sc_stage(
        padf(d2.reshape(N_EDGES)), padf(c12.reshape(N_EDGES)),
        padf(c02.reshape(N_EDGES)),
        padf(r[:, 0]), padf(r[:, 1]), padf(r[:, 2]),
        padf(src), padf(dst), coefs, zn, zf)

    total_energy = (esum[0] + psum[0, 0]).reshape(1)
    forces = fout[:N_NODES, :3]
    return (total_energy, forces)


# speedup vs baseline: 11.5160x; 1.2238x over previous
import functools

import jax
import jax.numpy as jnp
import numpy as np
from jax import lax
from jax.experimental import pallas as pl
from jax.experimental.pallas import tpu as pltpu
from jax.experimental.pallas import tpu_sc as plsc

NBASIS = 128
CUTOFF = 6.0
N_NODES = 10000
N_EDGES = 320000

_DELTA = CUTOFF / (NBASIS - 1)
_GAMMA = 1.0 / _DELTA
_PI = float(np.pi)

_EPAD = 327680
_ROWS = 16
_GRID = _EPAD // (_ROWS * 128)


def _edge_body(w4_ref, rx_ref, ry_ref, rz_ref,
               d_ref, c1_ref, c0_ref, ps_ref):
    rx = rx_ref[0]
    ry = ry_ref[0]
    rz = rz_ref[0]
    l2 = rx * rx + ry * ry + rz * rz
    mask = l2 > 0.0
    l2s = jnp.maximum(l2, 1e-30)
    inv_l = lax.rsqrt(l2s)
    bl = l2s * inv_l
    th = (_PI / CUTOFF) * bl
    fc = 0.5 + 0.5 * jnp.cos(th)
    fcp = (-_PI / (2.0 * CUTOFF)) * jnp.sin(th)

    w4 = w4_ref[...]
    ccol = lax.broadcasted_iota(jnp.int32, (16, 1), 0).astype(jnp.float32) * (
        8.0 * _DELTA)

    d_rows, c1_rows, c0_rows = [], [], []
    pacc = jnp.zeros((1, 128), jnp.float32)
    for s in range(_ROWS):
        blr = bl[s:s + 1, :]
        t0 = blr - ccol
        anchor = jnp.exp((-_GAMMA) * (t0 * t0))
        w = jnp.exp(2.0 * t0)
        rows = [anchor]
        x = anchor
        for _ in range(7):
            x = x * w
            rows.append(x)
        e = jnp.concatenate(rows, axis=0)
        r4 = lax.dot_general(w4, e, (((1,), (0,)), ((), ())),
                             precision=lax.Precision.HIGHEST,
                             preferred_element_type=jnp.float32)
        sa, sac, sb, sbc = (r4[0:1], r4[1:2], r4[2:3], r4[3:4])

        fcr = fc[s:s + 1]
        fcpr = fcp[s:s + 1]
        invr = inv_l[s:s + 1]
        mr = mask[s:s + 1]
        ap = (-2.0 * _GAMMA) * (blr * sa - sac)
        bp = (-2.0 * _GAMMA) * (blr * sb - sbc)
        d = jnp.where(mr, 0.5 * sa * fcr, 0.0)
        c1 = jnp.where(mr, 0.5 * (ap * fcr + sa * fcpr) * invr, 0.0)
        p = jnp.where(mr, 0.5 * sb * fcr * invr, 0.0)
        c0 = jnp.where(mr,
                       ((bp * fcr + sb * fcpr) * 0.5 * invr - p * invr) * invr,
                       0.0)
        d_rows.append(d)
        c1_rows.append(c1)
        c0_rows.append(c0)
        pacc = pacc + p

    d_ref[...] = jnp.stack(d_rows, axis=0).reshape(1, _ROWS, 128)
    c1_ref[...] = jnp.stack(c1_rows, axis=0).reshape(1, _ROWS, 128)
    c0_ref[...] = jnp.stack(c0_rows, axis=0).reshape(1, _ROWS, 128)

    @pl.when(pl.program_id(0) == 0)
    def _():
        ps_ref[0, 0] = 0.0

    ps_ref[0, 0] += jnp.sum(pacc)


def _edge_stage(w4, rx2, ry2, rz2):
    blk = pl.BlockSpec((1, _ROWS, 128), lambda i: (i, 0, 0))
    return pl.pallas_call(
        _edge_body,
        grid=(_GRID,),
        in_specs=[
            pl.BlockSpec((4, 128), lambda i: (0, 0)),
            blk, blk, blk,
        ],
        out_specs=[
            blk, blk, blk,
            pl.BlockSpec((1, 1), lambda i: (0, 0), memory_space=pltpu.SMEM),
        ],
        out_shape=[
            jax.ShapeDtypeStruct((_GRID, _ROWS, 128), jnp.float32),
            jax.ShapeDtypeStruct((_GRID, _ROWS, 128), jnp.float32),
            jax.ShapeDtypeStruct((_GRID, _ROWS, 128), jnp.float32),
            jax.ShapeDtypeStruct((1, 1), jnp.float32),
        ],
    )(w4, rx2, ry2, rz2)


_NT = 16
_EPW = _EPAD // _NT
_NSUB = 4
_CH = _EPW // _NSUB
_NPAD = 10240
_NPT = _NPAD // _NT


def _rsqrt_newton(x):
    m = x
    sc = jnp.full((16,), 1.0, jnp.float32)
    for _ in range(16):
        big = m >= 4.0
        m = jnp.where(big, m * 0.25, m)
        sc = jnp.where(big, sc * 0.5, sc)
    for _ in range(32):
        small = m < 1.0
        m = jnp.where(small, m * 4.0, m)
        sc = jnp.where(small, sc * 2.0, sc)
    y = 1.30880086 + m * (-0.39516662 + m * 0.04939814)
    for _ in range(3):
        y = y * (1.5 - (0.5 * m) * y * y)
    return y * sc


def _sc_body(d_hbm, c1_hbm, c0_hbm, rx_hbm, ry_hbm, rz_hbm,
             src_hbm, dst_hbm, coef_hbm, zn_hbm, zf_hbm,
             fout_hbm, esum_hbm,
             rho_sh, fp_sh, fpos_sh, fneg_sh, es_sh,
             idxd, idxs, v0, v1, vx, vy, vz, dxyz,
             fp_loc, rho_loc, fp_part, coef_v, va, vb, es_loc):
    wid = lax.axis_index("s")
    ebase = wid * _EPW
    nbase = wid * _NPT

    pltpu.sync_copy(zn_hbm, rho_loc)
    pltpu.sync_copy(rho_loc, rho_sh.at[pl.ds(nbase, _NPT)])
    pltpu.sync_copy(zf_hbm, va)
    pltpu.sync_copy(va, fpos_sh.at[pl.ds(nbase, _NPT)])
    pltpu.sync_copy(va, fneg_sh.at[pl.ds(nbase, _NPT)])
    for k in range(_CH // _NPT):
        pltpu.sync_copy(zf_hbm, dxyz.at[pl.ds(k * _NPT, _NPT)])
    pltpu.sync_copy(coef_hbm, coef_v)
    plsc.subcore_barrier()

    for s in range(_NSUB):
        off = ebase + s * _CH
        pltpu.sync_copy(d_hbm.at[pl.ds(off, _CH)], v0)
        pltpu.sync_copy(dst_hbm.at[pl.ds(off, _CH)], idxd)
        pltpu.sync_copy(v0, rho_sh.at[idxd], add=True)
    plsc.subcore_barrier()

    pltpu.sync_copy(rho_sh.at[pl.ds(nbase, _NPT)], rho_loc)
    cf = [coef_v[j] for j in range(5)]
    gf = [coef_v[5 + j] for j in range(5)]

    def node_step(i, facc):
        o = i * 16
        x = rho_loc[pl.ds(o, 16)]
        y = _rsqrt_newton(x)
        s = x * y
        fval = cf[0] * s + x * (cf[1] + x * (cf[2] + x * (cf[3] + x * cf[4])))
        fp = gf[0] * y + gf[1] + x * (gf[2] + x * (gf[3] + x * gf[4]))
        fp_part[pl.ds(o, 16)] = fp
        return facc + fval

    facc = lax.fori_loop(0, _NPT // 16, node_step, jnp.zeros((16,), jnp.float32))
    pltpu.sync_copy(fp_part, fp_sh.at[pl.ds(nbase, _NPT)])
    es_loc[0] = facc
    pltpu.sync_copy(es_loc.at[0], es_sh.at[wid])
    plsc.subcore_barrier()

    pltpu.sync_copy(fp_sh, fp_loc)
    iota = lax.iota(jnp.int32, 16)
    col0 = jnp.zeros((16,), jnp.int32)
    col1 = jnp.full((16,), 1, jnp.int32)
    col2 = jnp.full((16,), 2, jnp.int32)
    for s in range(_NSUB):
        off = ebase + s * _CH
        pltpu.sync_copy(c1_hbm.at[pl.ds(off, _CH)], v0)
        pltpu.sync_copy(c0_hbm.at[pl.ds(off, _CH)], v1)
        pltpu.sync_copy(dst_hbm.at[pl.ds(off, _CH)], idxd)
        pltpu.sync_copy(src_hbm.at[pl.ds(off, _CH)], idxs)
        pltpu.sync_copy(rx_hbm.at[pl.ds(off, _CH)], vx)
        pltpu.sync_copy(ry_hbm.at[pl.ds(off, _CH)], vy)
        pltpu.sync_copy(rz_hbm.at[pl.ds(off, _CH)], vz)

        def edge_step(j, _):
            o = j * 16
            dv = idxd[pl.ds(o, 16)]
            fpd = plsc.load_gather(fp_loc, ([dv]))
            g = fpd * v0[pl.ds(o, 16)] + v1[pl.ds(o, 16)]
            rows = iota + o
            plsc.store_scatter(dxyz, ([rows, col0]),
                               g * vx[pl.ds(o, 16)])
            plsc.store_scatter(dxyz, ([rows, col1]),
                               g * vy[pl.ds(o, 16)])
            plsc.store_scatter(dxyz, ([rows, col2]),
                               g * vz[pl.ds(o, 16)])
            return 0

        lax.fori_loop(0, _CH // 16, edge_step, 0)
        pltpu.sync_copy(dxyz, fpos_sh.at[idxs], add=True)
        pltpu.sync_copy(dxyz, fneg_sh.at[idxd], add=True)
    plsc.subcore_barrier()

    pltpu.sync_copy(fpos_sh.at[pl.ds(nbase, _NPT)], va)
    pltpu.sync_copy(fneg_sh.at[pl.ds(nbase, _NPT)], vb)

    def sub_step(m, _):
        c = m // (_NPT // 16)
        rows = iota + 16 * (m % (_NPT // 16))
        colv = jnp.full((16,), 1, jnp.int32) * c
        a = plsc.load_gather(va, ([rows, colv]))
        b = plsc.load_gather(vb, ([rows, colv]))
        plsc.store_scatter(va, ([rows, colv]), a - b)
        return 0

    lax.fori_loop(0, 3 * (_NPT // 16), sub_step, 0)
    pltpu.sync_copy(va, fout_hbm.at[pl.ds(nbase, _NPT)])

    @pl.when(wid == 0)
    def _():
        pltpu.sync_copy(es_sh, es_loc)
        acc = es_loc[0]
        for t in range(1, _NT):
            acc = acc + es_loc[t]
        tot = lax.broadcast(jnp.sum(acc, axis=0), (16,))
        es_loc[0] = tot
        pltpu.sync_copy(es_loc.at[0], esum_hbm)


def _sc_stage(d_e, c1_e, c0_e, rx, ry, rz, src, dst, coefs, zn, zf):
    mesh = plsc.VectorSubcoreMesh(core_axis_name="c", subcore_axis_name="s",
                                  num_cores=1, num_subcores=_NT)
    f32 = jnp.float32
    run = pl.kernel(
        _sc_body,
        out_type=[
            jax.ShapeDtypeStruct((_NPAD, 8), f32),
            jax.ShapeDtypeStruct((16,), f32),
        ],
        mesh=mesh,
        compiler_params=pltpu.CompilerParams(needs_layout_passes=False,
                                             use_tc_tiling_on_sc=False),
        scratch_types=[
            pltpu.VMEM_SHARED((_NPAD,), f32),
            pltpu.VMEM_SHARED((_NPAD,), f32),
            pltpu.VMEM_SHARED((_NPAD, 8), f32),
            pltpu.VMEM_SHARED((_NPAD, 8), f32),
            pltpu.VMEM_SHARED((_NT, 16), f32),
            pltpu.VMEM((_CH,), jnp.int32),
            pltpu.VMEM((_CH,), jnp.int32),
            pltpu.VMEM((_CH,), f32),
            pltpu.VMEM((_CH,), f32),
            pltpu.VMEM((_CH,), f32),
            pltpu.VMEM((_CH,), f32),
            pltpu.VMEM((_CH,), f32),
            pltpu.VMEM((_CH, 8), f32),
            pltpu.VMEM((_NPAD,), f32),
            pltpu.VMEM((_NPT,), f32),
            pltpu.VMEM((_NPT,), f32),
            pltpu.VMEM((10, 16), f32),
            pltpu.VMEM((_NPT, 8), f32),
            pltpu.VMEM((_NPT, 8), f32),
            pltpu.VMEM((_NT, 16), f32),
        ],
    )
    return run(d_e, c1_e, c0_e, rx, ry, rz, src, dst, coefs, zn, zf)


_POWERS = np.concatenate([[0.5], 1.0 + np.arange(4)]).astype(np.float32)
_SF = np.concatenate(
    [[2.0], 1.0 / 10.0 ** np.cumsum(np.log10(1.0 + np.arange(4)))]
).astype(np.float32)


@jax.jit
def kernel(r, edge_index, phi_density, phi_pair, emb_weights):
    sp = jax.nn.softplus(phi_density)
    pp = phi_pair

    rr = np.arange(128)
    karr = 8 * (rr % 16) + rr // 16
    qarr = jnp.asarray(np.exp(-((rr // 16) ** 2) * _DELTA).astype(np.float32))
    cen = jnp.asarray((_DELTA * np.arange(128)).astype(np.float32))
    spk = sp[karr] * qarr
    ppk = pp[karr] * qarr
    ck = cen[karr]
    w4 = jnp.stack([spk, spk * ck, ppk, ppk * ck], axis=0)

    npad = _EPAD - N_EDGES
    rxp = jnp.pad(r[:, 0], (0, npad))
    ryp = jnp.pad(r[:, 1], (0, npad))
    rzp = jnp.pad(r[:, 2], (0, npad))

    d2, c12, c02, psum = _edge_stage(
        w4,
        rxp.reshape(_GRID, _ROWS, 128),
        ryp.reshape(_GRID, _ROWS, 128),
        rzp.reshape(_GRID, _ROWS, 128))

    cf = emb_weights * jnp.asarray(_SF)
    gfc = cf * jnp.asarray(_POWERS)
    coefs = jnp.broadcast_to(
        jnp.concatenate([cf, gfc])[:, None], (10, 16)).astype(jnp.float32)

    src_i = jnp.pad(edge_index[0].astype(jnp.int32), (0, npad))
    dst_i = jnp.pad(edge_index[1].astype(jnp.int32), (0, npad))
    zn = jnp.zeros((_NPT,), jnp.float32)
    zf = jnp.zeros((_NPT, 8), jnp.float32)

    fout, esum = _sc_stage(
        d2.reshape(_EPAD), c12.reshape(_EPAD), c02.reshape(_EPAD),
        rxp, ryp, rzp, src_i, dst_i, coefs, zn, zf)

    total_energy = (esum[0] + psum[0, 0]).reshape(1)
    forces = fout[:N_NODES, :3]
    return (total_energy, forces)


# speedup vs baseline: 12.0314x; 1.0448x over previous
import functools

import jax
import jax.numpy as jnp
import numpy as np
from jax import lax
from jax.experimental import pallas as pl
from jax.experimental.pallas import tpu as pltpu
from jax.experimental.pallas import tpu_sc as plsc

NBASIS = 128
CUTOFF = 6.0
N_NODES = 10000
N_EDGES = 320000

_DELTA = CUTOFF / (NBASIS - 1)
_GAMMA = 1.0 / _DELTA
_PI = float(np.pi)

_EPAD = 327680
_ROWS = 16
_GRID = _EPAD // (_ROWS * 128)


def _edge_body(w4_ref, rx_ref, ry_ref, rz_ref,
               d_ref, c1_ref, c0_ref, ps_ref):
    rx = rx_ref[0]
    ry = ry_ref[0]
    rz = rz_ref[0]
    l2 = rx * rx + ry * ry + rz * rz
    mask = l2 > 0.0
    l2s = jnp.maximum(l2, 1e-30)
    inv_l = lax.rsqrt(l2s)
    bl = l2s * inv_l
    th = (_PI / CUTOFF) * bl
    fc = 0.5 + 0.5 * jnp.cos(th)
    fcp = (-_PI / (2.0 * CUTOFF)) * jnp.sin(th)

    w4 = w4_ref[...]
    ccol = lax.broadcasted_iota(jnp.int32, (16, 1), 0).astype(jnp.float32) * (
        8.0 * _DELTA)

    d_rows, c1_rows, c0_rows = [], [], []
    pacc = jnp.zeros((1, 128), jnp.float32)
    for s in range(_ROWS):
        blr = bl[s:s + 1, :]
        t0 = blr - ccol
        anchor = jnp.exp((-_GAMMA) * (t0 * t0))
        w = jnp.exp(2.0 * t0)
        rows = [anchor]
        x = anchor
        for _ in range(7):
            x = x * w
            rows.append(x)
        e = jnp.concatenate(rows, axis=0)
        r4 = lax.dot_general(w4, e, (((1,), (0,)), ((), ())),
                             precision=lax.Precision.HIGHEST,
                             preferred_element_type=jnp.float32)
        sa, sac, sb, sbc = (r4[0:1], r4[1:2], r4[2:3], r4[3:4])

        fcr = fc[s:s + 1]
        fcpr = fcp[s:s + 1]
        invr = inv_l[s:s + 1]
        mr = mask[s:s + 1]
        ap = (-2.0 * _GAMMA) * (blr * sa - sac)
        bp = (-2.0 * _GAMMA) * (blr * sb - sbc)
        d = jnp.where(mr, 0.5 * sa * fcr, 0.0)
        c1 = jnp.where(mr, 0.5 * (ap * fcr + sa * fcpr) * invr, 0.0)
        p = jnp.where(mr, 0.5 * sb * fcr * invr, 0.0)
        c0 = jnp.where(mr,
                       ((bp * fcr + sb * fcpr) * 0.5 * invr - p * invr) * invr,
                       0.0)
        d_rows.append(d)
        c1_rows.append(c1)
        c0_rows.append(c0)
        pacc = pacc + p

    d_ref[...] = jnp.stack(d_rows, axis=0).reshape(1, _ROWS, 128)
    c1_ref[...] = jnp.stack(c1_rows, axis=0).reshape(1, _ROWS, 128)
    c0_ref[...] = jnp.stack(c0_rows, axis=0).reshape(1, _ROWS, 128)

    @pl.when(pl.program_id(0) == 0)
    def _():
        ps_ref[0, 0] = 0.0

    ps_ref[0, 0] += jnp.sum(pacc)


def _edge_stage(w4, rx2, ry2, rz2):
    blk = pl.BlockSpec((1, _ROWS, 128), lambda i: (i, 0, 0))
    return pl.pallas_call(
        _edge_body,
        grid=(_GRID,),
        in_specs=[
            pl.BlockSpec((4, 128), lambda i: (0, 0)),
            blk, blk, blk,
        ],
        out_specs=[
            blk, blk, blk,
            pl.BlockSpec((1, 1), lambda i: (0, 0), memory_space=pltpu.SMEM),
        ],
        out_shape=[
            jax.ShapeDtypeStruct((_GRID, _ROWS, 128), jnp.float32),
            jax.ShapeDtypeStruct((_GRID, _ROWS, 128), jnp.float32),
            jax.ShapeDtypeStruct((_GRID, _ROWS, 128), jnp.float32),
            jax.ShapeDtypeStruct((1, 1), jnp.float32),
        ],
    )(w4, rx2, ry2, rz2)


_NT = 16
_EPW = _EPAD // _NT
_NSUB = 4
_CH = _EPW // _NSUB
_NPAD = 10240
_NPT = _NPAD // _NT


def _rsqrt_newton(x):
    m = x
    sc = jnp.full((16,), 1.0, jnp.float32)
    for _ in range(16):
        big = m >= 4.0
        m = jnp.where(big, m * 0.25, m)
        sc = jnp.where(big, sc * 0.5, sc)
    for _ in range(32):
        small = m < 1.0
        m = jnp.where(small, m * 4.0, m)
        sc = jnp.where(small, sc * 2.0, sc)
    y = 1.30880086 + m * (-0.39516662 + m * 0.04939814)
    for _ in range(3):
        y = y * (1.5 - (0.5 * m) * y * y)
    return y * sc


def _sc_body(d_hbm, c1_hbm, c0_hbm, rx_hbm, ry_hbm, rz_hbm,
             src_hbm, dst_hbm, coef_hbm, zn_hbm, zf_hbm,
             fout_hbm, esum_hbm,
             rho_sh, fp_sh, fpos_sh, fneg_sh, es_sh,
             idxd, idxs, v0, v1, vx, vy, vz, dxyz,
             fp_loc, rho_loc, fp_part, coef_v, va, vb, es_loc, sem):
    wid = lax.axis_index("s")
    ebase = wid * _EPW
    nbase = wid * _NPT

    pltpu.sync_copy(zn_hbm, rho_loc)
    pltpu.sync_copy(rho_loc, rho_sh.at[pl.ds(nbase, _NPT)])
    pltpu.sync_copy(zf_hbm, va)
    pltpu.sync_copy(va, fpos_sh.at[pl.ds(nbase, _NPT)])
    pltpu.sync_copy(va, fneg_sh.at[pl.ds(nbase, _NPT)])
    for k in range(_CH // _NPT):
        pltpu.sync_copy(zf_hbm, dxyz.at[pl.ds(k * _NPT, _NPT)])
    pltpu.sync_copy(coef_hbm, coef_v)
    plsc.subcore_barrier()

    for s in range(_NSUB):
        off = ebase + s * _CH
        cps = [pltpu.async_copy(d_hbm.at[pl.ds(off, _CH)], v0, sem),
               pltpu.async_copy(dst_hbm.at[pl.ds(off, _CH)], idxd, sem)]
        for cp in cps:
            cp.wait()
        pltpu.sync_copy(v0, rho_sh.at[idxd], add=True)
    plsc.subcore_barrier()

    pltpu.sync_copy(rho_sh.at[pl.ds(nbase, _NPT)], rho_loc)
    cf = [coef_v[j] for j in range(5)]
    gf = [coef_v[5 + j] for j in range(5)]

    def node_step(i, facc):
        o = i * 16
        x = rho_loc[pl.ds(o, 16)]
        y = _rsqrt_newton(x)
        s = x * y
        fval = cf[0] * s + x * (cf[1] + x * (cf[2] + x * (cf[3] + x * cf[4])))
        fp = gf[0] * y + gf[1] + x * (gf[2] + x * (gf[3] + x * gf[4]))
        fp_part[pl.ds(o, 16)] = fp
        return facc + fval

    facc = lax.fori_loop(0, _NPT // 16, node_step, jnp.zeros((16,), jnp.float32))
    pltpu.sync_copy(fp_part, fp_sh.at[pl.ds(nbase, _NPT)])
    es_loc[0] = facc
    pltpu.sync_copy(es_loc.at[0], es_sh.at[wid])
    plsc.subcore_barrier()

    pltpu.sync_copy(fp_sh, fp_loc)
    iota = lax.iota(jnp.int32, 16)
    col0 = jnp.zeros((16,), jnp.int32)
    col1 = jnp.full((16,), 1, jnp.int32)
    col2 = jnp.full((16,), 2, jnp.int32)
    for s in range(_NSUB):
        off = ebase + s * _CH
        cps = [pltpu.async_copy(c1_hbm.at[pl.ds(off, _CH)], v0, sem),
               pltpu.async_copy(c0_hbm.at[pl.ds(off, _CH)], v1, sem),
               pltpu.async_copy(dst_hbm.at[pl.ds(off, _CH)], idxd, sem),
               pltpu.async_copy(src_hbm.at[pl.ds(off, _CH)], idxs, sem),
               pltpu.async_copy(rx_hbm.at[pl.ds(off, _CH)], vx, sem),
               pltpu.async_copy(ry_hbm.at[pl.ds(off, _CH)], vy, sem),
               pltpu.async_copy(rz_hbm.at[pl.ds(off, _CH)], vz, sem)]
        for cp in cps:
            cp.wait()

        def edge_step(j, _):
            o = j * 16
            dv = idxd[pl.ds(o, 16)]
            fpd = plsc.load_gather(fp_loc, ([dv]))
            g = fpd * v0[pl.ds(o, 16)] + v1[pl.ds(o, 16)]
            rows = iota + o
            plsc.store_scatter(dxyz, ([rows, col0]),
                               g * vx[pl.ds(o, 16)])
            plsc.store_scatter(dxyz, ([rows, col1]),
                               g * vy[pl.ds(o, 16)])
            plsc.store_scatter(dxyz, ([rows, col2]),
                               g * vz[pl.ds(o, 16)])
            return 0

        lax.fori_loop(0, _CH // 16, edge_step, 0)
        pltpu.sync_copy(dxyz, fpos_sh.at[idxs], add=True)
        pltpu.sync_copy(dxyz, fneg_sh.at[idxd], add=True)
    plsc.subcore_barrier()

    pltpu.sync_copy(fpos_sh.at[pl.ds(nbase, _NPT)], va)
    pltpu.sync_copy(fneg_sh.at[pl.ds(nbase, _NPT)], vb)

    def sub_step(m, _):
        c = m // (_NPT // 16)
        rows = iota + 16 * (m % (_NPT // 16))
        colv = jnp.full((16,), 1, jnp.int32) * c
        a = plsc.load_gather(va, ([rows, colv]))
        b = plsc.load_gather(vb, ([rows, colv]))
        plsc.store_scatter(va, ([rows, colv]), a - b)
        return 0

    lax.fori_loop(0, 3 * (_NPT // 16), sub_step, 0)
    pltpu.sync_copy(va, fout_hbm.at[pl.ds(nbase, _NPT)])

    @pl.when(wid == 0)
    def _():
        pltpu.sync_copy(es_sh, es_loc)
        acc = es_loc[0]
        for t in range(1, _NT):
            acc = acc + es_loc[t]
        tot = lax.broadcast(jnp.sum(acc, axis=0), (16,))
        es_loc[0] = tot
        pltpu.sync_copy(es_loc.at[0], esum_hbm)


def _sc_stage(d_e, c1_e, c0_e, rx, ry, rz, src, dst, coefs, zn, zf):
    mesh = plsc.VectorSubcoreMesh(core_axis_name="c", subcore_axis_name="s",
                                  num_cores=1, num_subcores=_NT)
    f32 = jnp.float32
    run = pl.kernel(
        _sc_body,
        out_type=[
            jax.ShapeDtypeStruct((_NPAD, 8), f32),
            jax.ShapeDtypeStruct((16,), f32),
        ],
        mesh=mesh,
        compiler_params=pltpu.CompilerParams(needs_layout_passes=False,
                                             use_tc_tiling_on_sc=False),
        scratch_types=[
            pltpu.VMEM_SHARED((_NPAD,), f32),
            pltpu.VMEM_SHARED((_NPAD,), f32),
            pltpu.VMEM_SHARED((_NPAD, 8), f32),
            pltpu.VMEM_SHARED((_NPAD, 8), f32),
            pltpu.VMEM_SHARED((_NT, 16), f32),
            pltpu.VMEM((_CH,), jnp.int32),
            pltpu.VMEM((_CH,), jnp.int32),
            pltpu.VMEM((_CH,), f32),
            pltpu.VMEM((_CH,), f32),
            pltpu.VMEM((_CH,), f32),
            pltpu.VMEM((_CH,), f32),
            pltpu.VMEM((_CH,), f32),
            pltpu.VMEM((_CH, 8), f32),
            pltpu.VMEM((_NPAD,), f32),
            pltpu.VMEM((_NPT,), f32),
            pltpu.VMEM((_NPT,), f32),
            pltpu.VMEM((10, 16), f32),
            pltpu.VMEM((_NPT, 8), f32),
            pltpu.VMEM((_NPT, 8), f32),
            pltpu.VMEM((_NT, 16), f32),
            pltpu.SemaphoreType.DMA,
        ],
    )
    return run(d_e, c1_e, c0_e, rx, ry, rz, src, dst, coefs, zn, zf)


_POWERS = np.concatenate([[0.5], 1.0 + np.arange(4)]).astype(np.float32)
_SF = np.concatenate(
    [[2.0], 1.0 / 10.0 ** np.cumsum(np.log10(1.0 + np.arange(4)))]
).astype(np.float32)


@jax.jit
def kernel(r, edge_index, phi_density, phi_pair, emb_weights):
    sp = jax.nn.softplus(phi_density)
    pp = phi_pair

    rr = np.arange(128)
    karr = 8 * (rr % 16) + rr // 16
    qarr = jnp.asarray(np.exp(-((rr // 16) ** 2) * _DELTA).astype(np.float32))
    cen = jnp.asarray((_DELTA * np.arange(128)).astype(np.float32))
    spk = sp[karr] * qarr
    ppk = pp[karr] * qarr
    ck = cen[karr]
    w4 = jnp.stack([spk, spk * ck, ppk, ppk * ck], axis=0)

    npad = _EPAD - N_EDGES
    rxp = jnp.pad(r[:, 0], (0, npad))
    ryp = jnp.pad(r[:, 1], (0, npad))
    rzp = jnp.pad(r[:, 2], (0, npad))

    d2, c12, c02, psum = _edge_stage(
        w4,
        rxp.reshape(_GRID, _ROWS, 128),
        ryp.reshape(_GRID, _ROWS, 128),
        rzp.reshape(_GRID, _ROWS, 128))

    cf = emb_weights * jnp.asarray(_SF)
    gfc = cf * jnp.asarray(_POWERS)
    coefs = jnp.broadcast_to(
        jnp.concatenate([cf, gfc])[:, None], (10, 16)).astype(jnp.float32)

    src_i = jnp.pad(edge_index[0].astype(jnp.int32), (0, npad))
    dst_i = jnp.pad(edge_index[1].astype(jnp.int32), (0, npad))
    zn = jnp.zeros((_NPT,), jnp.float32)
    zf = jnp.zeros((_NPT, 8), jnp.float32)

    fout, esum = _sc_stage(
        d2.reshape(_EPAD), c12.reshape(_EPAD), c02.reshape(_EPAD),
        rxp, ryp, rzp, src_i, dst_i, coefs, zn, zf)

    total_energy = (esum[0] + psum[0, 0]).reshape(1)
    forces = fout[:N_NODES, :3]
    return (total_energy, forces)


# speedup vs baseline: 12.0637x; 1.0027x over previous
import functools

import jax
import jax.numpy as jnp
import numpy as np
from jax import lax
from jax.experimental import pallas as pl
from jax.experimental.pallas import tpu as pltpu
from jax.experimental.pallas import tpu_sc as plsc

NBASIS = 128
CUTOFF = 6.0
N_NODES = 10000
N_EDGES = 320000

_DELTA = CUTOFF / (NBASIS - 1)
_GAMMA = 1.0 / _DELTA
_PI = float(np.pi)

_EPAD = 327680
_ROWS = 16
_GRID = _EPAD // (_ROWS * 128)


def _edge_body(w4_ref, rx_ref, ry_ref, rz_ref,
               d_ref, c1_ref, c0_ref, ps_ref):
    rx = rx_ref[0]
    ry = ry_ref[0]
    rz = rz_ref[0]
    l2 = rx * rx + ry * ry + rz * rz
    mask = l2 > 0.0
    l2s = jnp.maximum(l2, 1e-30)
    inv_l = lax.rsqrt(l2s)
    bl = l2s * inv_l
    th = (_PI / CUTOFF) * bl
    fc = 0.5 + 0.5 * jnp.cos(th)
    fcp = (-_PI / (2.0 * CUTOFF)) * jnp.sin(th)

    w4 = w4_ref[...]
    ccol = lax.broadcasted_iota(jnp.int32, (16, 1), 0).astype(jnp.float32) * (
        8.0 * _DELTA)

    sa_rows, sac_rows, sb_rows, sbc_rows = [], [], [], []
    for s in range(_ROWS):
        blr = bl[s:s + 1, :]
        t0 = blr - ccol
        anchor = jnp.exp((-_GAMMA) * (t0 * t0))
        w = jnp.exp(2.0 * t0)
        rows = [anchor]
        x = anchor
        for _ in range(7):
            x = x * w
            rows.append(x)
        e = jnp.concatenate(rows, axis=0)
        r4 = lax.dot_general(w4, e, (((1,), (0,)), ((), ())),
                             precision=lax.Precision.HIGHEST,
                             preferred_element_type=jnp.float32)
        sa_rows.append(r4[0:1])
        sac_rows.append(r4[1:2])
        sb_rows.append(r4[2:3])
        sbc_rows.append(r4[3:4])

    sa = jnp.concatenate(sa_rows, axis=0)
    sac = jnp.concatenate(sac_rows, axis=0)
    sb = jnp.concatenate(sb_rows, axis=0)
    sbc = jnp.concatenate(sbc_rows, axis=0)

    ap = (-2.0 * _GAMMA) * (bl * sa - sac)
    bp = (-2.0 * _GAMMA) * (bl * sb - sbc)
    zero = jnp.zeros_like(bl)
    d = jnp.where(mask, 0.5 * sa * fc, zero)
    c1 = jnp.where(mask, 0.5 * (ap * fc + sa * fcp) * inv_l, zero)
    p = jnp.where(mask, 0.5 * sb * fc * inv_l, zero)
    c0 = jnp.where(mask, ((bp * fc + sb * fcp) * 0.5 * inv_l - p * inv_l)
                   * inv_l, zero)

    d_ref[...] = d.reshape(1, _ROWS, 128)
    c1_ref[...] = c1.reshape(1, _ROWS, 128)
    c0_ref[...] = c0.reshape(1, _ROWS, 128)

    @pl.when(pl.program_id(0) == 0)
    def _():
        ps_ref[0, 0] = 0.0

    ps_ref[0, 0] += jnp.sum(p)


def _edge_stage(w4, rx2, ry2, rz2):
    blk = pl.BlockSpec((1, _ROWS, 128), lambda i: (i, 0, 0))
    return pl.pallas_call(
        _edge_body,
        grid=(_GRID,),
        in_specs=[
            pl.BlockSpec((4, 128), lambda i: (0, 0)),
            blk, blk, blk,
        ],
        out_specs=[
            blk, blk, blk,
            pl.BlockSpec((1, 1), lambda i: (0, 0), memory_space=pltpu.SMEM),
        ],
        out_shape=[
            jax.ShapeDtypeStruct((_GRID, _ROWS, 128), jnp.float32),
            jax.ShapeDtypeStruct((_GRID, _ROWS, 128), jnp.float32),
            jax.ShapeDtypeStruct((_GRID, _ROWS, 128), jnp.float32),
            jax.ShapeDtypeStruct((1, 1), jnp.float32),
        ],
    )(w4, rx2, ry2, rz2)


_NT = 16
_EPW = _EPAD // _NT
_NSUB = 4
_CH = _EPW // _NSUB
_NPAD = 10240
_NPT = _NPAD // _NT


def _rsqrt_newton(x):
    m = x
    sc = jnp.full((16,), 1.0, jnp.float32)
    for _ in range(16):
        big = m >= 4.0
        m = jnp.where(big, m * 0.25, m)
        sc = jnp.where(big, sc * 0.5, sc)
    for _ in range(32):
        small = m < 1.0
        m = jnp.where(small, m * 4.0, m)
        sc = jnp.where(small, sc * 2.0, sc)
    y = 1.30880086 + m * (-0.39516662 + m * 0.04939814)
    for _ in range(3):
        y = y * (1.5 - (0.5 * m) * y * y)
    return y * sc


def _sc_body(d_hbm, c1_hbm, c0_hbm, rx_hbm, ry_hbm, rz_hbm,
             src_hbm, dst_hbm, coef_hbm, zn_hbm, zf_hbm,
             fout_hbm, esum_hbm,
             rho_sh, fp_sh, fpos_sh, fneg_sh, es_sh,
             idxd, idxs, v0, v1, vx, vy, vz, dxyz,
             fp_loc, rho_loc, fp_part, coef_v, va, vb, es_loc, sem):
    wid = lax.axis_index("s")
    ebase = wid * _EPW
    nbase = wid * _NPT

    pltpu.sync_copy(zn_hbm, rho_loc)
    pltpu.sync_copy(rho_loc, rho_sh.at[pl.ds(nbase, _NPT)])
    pltpu.sync_copy(zf_hbm, va)
    pltpu.sync_copy(va, fpos_sh.at[pl.ds(nbase, _NPT)])
    pltpu.sync_copy(va, fneg_sh.at[pl.ds(nbase, _NPT)])
    for k in range(_CH // _NPT):
        pltpu.sync_copy(zf_hbm, dxyz.at[pl.ds(k * _NPT, _NPT)])
    pltpu.sync_copy(coef_hbm, coef_v)
    plsc.subcore_barrier()

    for s in range(_NSUB):
        off = ebase + s * _CH
        cps = [pltpu.async_copy(d_hbm.at[pl.ds(off, _CH)], v0, sem),
               pltpu.async_copy(dst_hbm.at[pl.ds(off, _CH)], idxd, sem)]
        for cp in cps:
            cp.wait()
        pltpu.sync_copy(v0, rho_sh.at[idxd], add=True)
    plsc.subcore_barrier()

    pltpu.sync_copy(rho_sh.at[pl.ds(nbase, _NPT)], rho_loc)
    cf = [coef_v[j] for j in range(5)]
    gf = [coef_v[5 + j] for j in range(5)]

    def node_step(i, facc):
        o = i * 16
        x = rho_loc[pl.ds(o, 16)]
        y = _rsqrt_newton(x)
        s = x * y
        fval = cf[0] * s + x * (cf[1] + x * (cf[2] + x * (cf[3] + x * cf[4])))
        fp = gf[0] * y + gf[1] + x * (gf[2] + x * (gf[3] + x * gf[4]))
        fp_part[pl.ds(o, 16)] = fp
        return facc + fval

    facc = lax.fori_loop(0, _NPT // 16, node_step, jnp.zeros((16,), jnp.float32))
    pltpu.sync_copy(fp_part, fp_sh.at[pl.ds(nbase, _NPT)])
    es_loc[0] = facc
    pltpu.sync_copy(es_loc.at[0], es_sh.at[wid])
    plsc.subcore_barrier()

    pltpu.sync_copy(fp_sh, fp_loc)
    iota = lax.iota(jnp.int32, 16)
    col0 = jnp.zeros((16,), jnp.int32)
    col1 = jnp.full((16,), 1, jnp.int32)
    col2 = jnp.full((16,), 2, jnp.int32)
    for s in range(_NSUB):
        off = ebase + s * _CH
        cps = [pltpu.async_copy(c1_hbm.at[pl.ds(off, _CH)], v0, sem),
               pltpu.async_copy(c0_hbm.at[pl.ds(off, _CH)], v1, sem),
               pltpu.async_copy(dst_hbm.at[pl.ds(off, _CH)], idxd, sem),
               pltpu.async_copy(src_hbm.at[pl.ds(off, _CH)], idxs, sem),
               pltpu.async_copy(rx_hbm.at[pl.ds(off, _CH)], vx, sem),
               pltpu.async_copy(ry_hbm.at[pl.ds(off, _CH)], vy, sem),
               pltpu.async_copy(rz_hbm.at[pl.ds(off, _CH)], vz, sem)]
        for cp in cps:
            cp.wait()

        def edge_step(j, _):
            o = j * 16
            dv = idxd[pl.ds(o, 16)]
            fpd = plsc.load_gather(fp_loc, ([dv]))
            g = fpd * v0[pl.ds(o, 16)] + v1[pl.ds(o, 16)]
            rows = iota + o
            plsc.store_scatter(dxyz, ([rows, col0]),
                               g * vx[pl.ds(o, 16)])
            plsc.store_scatter(dxyz, ([rows, col1]),
                               g * vy[pl.ds(o, 16)])
            plsc.store_scatter(dxyz, ([rows, col2]),
                               g * vz[pl.ds(o, 16)])
            return 0

        lax.fori_loop(0, _CH // 16, edge_step, 0)
        pltpu.sync_copy(dxyz, fpos_sh.at[idxs], add=True)
        pltpu.sync_copy(dxyz, fneg_sh.at[idxd], add=True)
    plsc.subcore_barrier()

    pltpu.sync_copy(fpos_sh.at[pl.ds(nbase, _NPT)], va)
    pltpu.sync_copy(fneg_sh.at[pl.ds(nbase, _NPT)], vb)

    def sub_step(m, _):
        c = m // (_NPT // 16)
        rows = iota + 16 * (m % (_NPT // 16))
        colv = jnp.full((16,), 1, jnp.int32) * c
        a = plsc.load_gather(va, ([rows, colv]))
        b = plsc.load_gather(vb, ([rows, colv]))
        plsc.store_scatter(va, ([rows, colv]), a - b)
        return 0

    lax.fori_loop(0, 3 * (_NPT // 16), sub_step, 0)
    pltpu.sync_copy(va, fout_hbm.at[pl.ds(nbase, _NPT)])

    @pl.when(wid == 0)
    def _():
        pltpu.sync_copy(es_sh, es_loc)
        acc = es_loc[0]
        for t in range(1, _NT):
            acc = acc + es_loc[t]
        tot = lax.broadcast(jnp.sum(acc, axis=0), (16,))
        es_loc[0] = tot
        pltpu.sync_copy(es_loc.at[0], esum_hbm)


def _sc_stage(d_e, c1_e, c0_e, rx, ry, rz, src, dst, coefs, zn, zf):
    mesh = plsc.VectorSubcoreMesh(core_axis_name="c", subcore_axis_name="s",
                                  num_cores=1, num_subcores=_NT)
    f32 = jnp.float32
    run = pl.kernel(
        _sc_body,
        out_type=[
            jax.ShapeDtypeStruct((_NPAD, 8), f32),
            jax.ShapeDtypeStruct((16,), f32),
        ],
        mesh=mesh,
        compiler_params=pltpu.CompilerParams(needs_layout_passes=False,
                                             use_tc_tiling_on_sc=False),
        scratch_types=[
            pltpu.VMEM_SHARED((_NPAD,), f32),
            pltpu.VMEM_SHARED((_NPAD,), f32),
            pltpu.VMEM_SHARED((_NPAD, 8), f32),
            pltpu.VMEM_SHARED((_NPAD, 8), f32),
            pltpu.VMEM_SHARED((_NT, 16), f32),
            pltpu.VMEM((_CH,), jnp.int32),
            pltpu.VMEM((_CH,), jnp.int32),
            pltpu.VMEM((_CH,), f32),
            pltpu.VMEM((_CH,), f32),
            pltpu.VMEM((_CH,), f32),
            pltpu.VMEM((_CH,), f32),
            pltpu.VMEM((_CH,), f32),
            pltpu.VMEM((_CH, 8), f32),
            pltpu.VMEM((_NPAD,), f32),
            pltpu.VMEM((_NPT,), f32),
            pltpu.VMEM((_NPT,), f32),
            pltpu.VMEM((10, 16), f32),
            pltpu.VMEM((_NPT, 8), f32),
            pltpu.VMEM((_NPT, 8), f32),
            pltpu.VMEM((_NT, 16), f32),
            pltpu.SemaphoreType.DMA,
        ],
    )
    return run(d_e, c1_e, c0_e, rx, ry, rz, src, dst, coefs, zn, zf)


_POWERS = np.concatenate([[0.5], 1.0 + np.arange(4)]).astype(np.float32)
_SF = np.concatenate(
    [[2.0], 1.0 / 10.0 ** np.cumsum(np.log10(1.0 + np.arange(4)))]
).astype(np.float32)


@jax.jit
def kernel(r, edge_index, phi_density, phi_pair, emb_weights):
    sp = jax.nn.softplus(phi_density)
    pp = phi_pair

    rr = np.arange(128)
    karr = 8 * (rr % 16) + rr // 16
    qarr = jnp.asarray(np.exp(-((rr // 16) ** 2) * _DELTA).astype(np.float32))
    cen = jnp.asarray((_DELTA * np.arange(128)).astype(np.float32))
    spk = sp[karr] * qarr
    ppk = pp[karr] * qarr
    ck = cen[karr]
    w4 = jnp.stack([spk, spk * ck, ppk, ppk * ck], axis=0)

    npad = _EPAD - N_EDGES
    rxp = jnp.pad(r[:, 0], (0, npad))
    ryp = jnp.pad(r[:, 1], (0, npad))
    rzp = jnp.pad(r[:, 2], (0, npad))

    d2, c12, c02, psum = _edge_stage(
        w4,
        rxp.reshape(_GRID, _ROWS, 128),
        ryp.reshape(_GRID, _ROWS, 128),
        rzp.reshape(_GRID, _ROWS, 128))

    cf = emb_weights * jnp.asarray(_SF)
    gfc = cf * jnp.asarray(_POWERS)
    coefs = jnp.broadcast_to(
        jnp.concatenate([cf, gfc])[:, None], (10, 16)).astype(jnp.float32)

    src_i = jnp.pad(edge_index[0].astype(jnp.int32), (0, npad))
    dst_i = jnp.pad(edge_index[1].astype(jnp.int32), (0, npad))
    zn = jnp.zeros((_NPT,), jnp.float32)
    zf = jnp.zeros((_NPT, 8), jnp.float32)

    fout, esum = _sc_stage(
        d2.reshape(_EPAD), c12.reshape(_EPAD), c02.reshape(_EPAD),
        rxp, ryp, rzp, src_i, dst_i, coefs, zn, zf)

    total_energy = (esum[0] + psum[0, 0]).reshape(1)
    forces = fout[:N_NODES, :3]
    return (total_energy, forces)


# speedup vs baseline: 14.1140x; 1.1700x over previous
import functools

import jax
import jax.numpy as jnp
import numpy as np
from jax import lax
from jax.experimental import pallas as pl
from jax.experimental.pallas import tpu as pltpu
from jax.experimental.pallas import tpu_sc as plsc

NBASIS = 128
CUTOFF = 6.0
N_NODES = 10000
N_EDGES = 320000

_DELTA = CUTOFF / (NBASIS - 1)
_GAMMA = 1.0 / _DELTA
_PI = float(np.pi)

_EPAD = 327680
_ROWS = 16
_GRID = _EPAD // (_ROWS * 128)


def _edge_body(w4_ref, rx_ref, ry_ref, rz_ref,
               d_ref, c1_ref, c0_ref, ps_ref):
    rx = rx_ref[0]
    ry = ry_ref[0]
    rz = rz_ref[0]
    l2 = rx * rx + ry * ry + rz * rz
    mask = l2 > 0.0
    l2s = jnp.maximum(l2, 1e-30)
    inv_l = lax.rsqrt(l2s)
    bl = l2s * inv_l
    th = (_PI / CUTOFF) * bl
    fc = 0.5 + 0.5 * jnp.cos(th)
    fcp = (-_PI / (2.0 * CUTOFF)) * jnp.sin(th)

    w4 = w4_ref[...]
    ccol = lax.broadcasted_iota(jnp.int32, (16, 1), 0).astype(jnp.float32) * (
        8.0 * _DELTA)

    sa_rows, sac_rows, sb_rows, sbc_rows = [], [], [], []
    _RPD = 4
    for g in range(_ROWS // _RPD):
        etiles = []
        for s in range(g * _RPD, (g + 1) * _RPD):
            blr = bl[s:s + 1, :]
            t0 = blr - ccol
            anchor = jnp.exp((-_GAMMA) * (t0 * t0))
            w = jnp.exp(2.0 * t0)
            rows = [anchor]
            x = anchor
            for _ in range(7):
                x = x * w
                rows.append(x)
            etiles.append(jnp.concatenate(rows, axis=0))
        ebig = jnp.concatenate(etiles, axis=1)
        r4 = lax.dot_general(w4, ebig, (((1,), (0,)), ((), ())),
                             precision=lax.Precision.HIGHEST,
                             preferred_element_type=jnp.float32)
        for q in range(_RPD):
            cs = q * 128
            sa_rows.append(r4[0:1, cs:cs + 128])
            sac_rows.append(r4[1:2, cs:cs + 128])
            sb_rows.append(r4[2:3, cs:cs + 128])
            sbc_rows.append(r4[3:4, cs:cs + 128])

    sa = jnp.concatenate(sa_rows, axis=0)
    sac = jnp.concatenate(sac_rows, axis=0)
    sb = jnp.concatenate(sb_rows, axis=0)
    sbc = jnp.concatenate(sbc_rows, axis=0)

    ap = (-2.0 * _GAMMA) * (bl * sa - sac)
    bp = (-2.0 * _GAMMA) * (bl * sb - sbc)
    zero = jnp.zeros_like(bl)
    d = jnp.where(mask, 0.5 * sa * fc, zero)
    c1 = jnp.where(mask, 0.5 * (ap * fc + sa * fcp) * inv_l, zero)
    p = jnp.where(mask, 0.5 * sb * fc * inv_l, zero)
    c0 = jnp.where(mask, ((bp * fc + sb * fcp) * 0.5 * inv_l - p * inv_l)
                   * inv_l, zero)

    d_ref[...] = d.reshape(1, _ROWS, 128)
    c1_ref[...] = c1.reshape(1, _ROWS, 128)
    c0_ref[...] = c0.reshape(1, _ROWS, 128)

    @pl.when(pl.program_id(0) == 0)
    def _():
        ps_ref[0, 0] = 0.0

    ps_ref[0, 0] += jnp.sum(p)


def _edge_stage(w4, rx2, ry2, rz2):
    blk = pl.BlockSpec((1, _ROWS, 128), lambda i: (i, 0, 0))
    return pl.pallas_call(
        _edge_body,
        grid=(_GRID,),
        in_specs=[
            pl.BlockSpec((4, 128), lambda i: (0, 0)),
            blk, blk, blk,
        ],
        out_specs=[
            blk, blk, blk,
            pl.BlockSpec((1, 1), lambda i: (0, 0), memory_space=pltpu.SMEM),
        ],
        out_shape=[
            jax.ShapeDtypeStruct((_GRID, _ROWS, 128), jnp.float32),
            jax.ShapeDtypeStruct((_GRID, _ROWS, 128), jnp.float32),
            jax.ShapeDtypeStruct((_GRID, _ROWS, 128), jnp.float32),
            jax.ShapeDtypeStruct((1, 1), jnp.float32),
        ],
    )(w4, rx2, ry2, rz2)


_NT = 16
_EPW = _EPAD // _NT
_NSUB = 4
_CH = _EPW // _NSUB
_NPAD = 10240
_NPT = _NPAD // _NT


def _rsqrt_newton(x):
    m = x
    sc = jnp.full((16,), 1.0, jnp.float32)
    for _ in range(16):
        big = m >= 4.0
        m = jnp.where(big, m * 0.25, m)
        sc = jnp.where(big, sc * 0.5, sc)
    for _ in range(32):
        small = m < 1.0
        m = jnp.where(small, m * 4.0, m)
        sc = jnp.where(small, sc * 2.0, sc)
    y = 1.30880086 + m * (-0.39516662 + m * 0.04939814)
    for _ in range(3):
        y = y * (1.5 - (0.5 * m) * y * y)
    return y * sc


def _sc_body(d_hbm, c1_hbm, c0_hbm, rx_hbm, ry_hbm, rz_hbm,
             src_hbm, dst_hbm, coef_hbm, zn_hbm, zf_hbm,
             fout_hbm, esum_hbm,
             rho_sh, fp_sh, fpos_sh, fneg_sh, es_sh,
             idxd, idxs, v0, v1, vx, vy, vz, dxyz,
             fp_loc, rho_loc, fp_part, coef_v, va, vb, es_loc, sem):
    wid = lax.axis_index("s")
    ebase = wid * _EPW
    nbase = wid * _NPT

    pltpu.sync_copy(zn_hbm, rho_loc)
    pltpu.sync_copy(rho_loc, rho_sh.at[pl.ds(nbase, _NPT)])
    pltpu.sync_copy(zf_hbm, va)
    pltpu.sync_copy(va, fpos_sh.at[pl.ds(nbase, _NPT)])
    pltpu.sync_copy(va, fneg_sh.at[pl.ds(nbase, _NPT)])
    for k in range(_CH // _NPT):
        pltpu.sync_copy(zf_hbm, dxyz.at[pl.ds(k * _NPT, _NPT)])
    pltpu.sync_copy(coef_hbm, coef_v)
    plsc.subcore_barrier()

    for s in range(_NSUB):
        off = ebase + s * _CH
        cps = [pltpu.async_copy(d_hbm.at[pl.ds(off, _CH)], v0, sem),
               pltpu.async_copy(dst_hbm.at[pl.ds(off, _CH)], idxd, sem)]
        for cp in cps:
            cp.wait()
        pltpu.sync_copy(v0, rho_sh.at[idxd], add=True)
    plsc.subcore_barrier()

    pltpu.sync_copy(rho_sh.at[pl.ds(nbase, _NPT)], rho_loc)
    cf = [coef_v[j] for j in range(5)]
    gf = [coef_v[5 + j] for j in range(5)]

    def node_step(i, facc):
        o = i * 16
        x = rho_loc[pl.ds(o, 16)]
        y = _rsqrt_newton(x)
        s = x * y
        fval = cf[0] * s + x * (cf[1] + x * (cf[2] + x * (cf[3] + x * cf[4])))
        fp = gf[0] * y + gf[1] + x * (gf[2] + x * (gf[3] + x * gf[4]))
        fp_part[pl.ds(o, 16)] = fp
        return facc + fval

    facc = lax.fori_loop(0, _NPT // 16, node_step, jnp.zeros((16,), jnp.float32))
    pltpu.sync_copy(fp_part, fp_sh.at[pl.ds(nbase, _NPT)])
    es_loc[0] = facc
    pltpu.sync_copy(es_loc.at[0], es_sh.at[wid])
    plsc.subcore_barrier()

    pltpu.sync_copy(fp_sh, fp_loc)
    iota = lax.iota(jnp.int32, 16)
    col0 = jnp.zeros((16,), jnp.int32)
    col1 = jnp.full((16,), 1, jnp.int32)
    col2 = jnp.full((16,), 2, jnp.int32)
    for s in range(_NSUB):
        off = ebase + s * _CH
        cps = [pltpu.async_copy(c1_hbm.at[pl.ds(off, _CH)], v0, sem),
               pltpu.async_copy(c0_hbm.at[pl.ds(off, _CH)], v1, sem),
               pltpu.async_copy(dst_hbm.at[pl.ds(off, _CH)], idxd, sem),
               pltpu.async_copy(src_hbm.at[pl.ds(off, _CH)], idxs, sem),
               pltpu.async_copy(rx_hbm.at[pl.ds(off, _CH)], vx, sem),
               pltpu.async_copy(ry_hbm.at[pl.ds(off, _CH)], vy, sem),
               pltpu.async_copy(rz_hbm.at[pl.ds(off, _CH)], vz, sem)]
        for cp in cps:
            cp.wait()

        def edge_step(j, _):
            o = j * 16
            dv = idxd[pl.ds(o, 16)]
            fpd = plsc.load_gather(fp_loc, ([dv]))
            g = fpd * v0[pl.ds(o, 16)] + v1[pl.ds(o, 16)]
            rows = iota + o
            plsc.store_scatter(dxyz, ([rows, col0]),
                               g * vx[pl.ds(o, 16)])
            plsc.store_scatter(dxyz, ([rows, col1]),
                               g * vy[pl.ds(o, 16)])
            plsc.store_scatter(dxyz, ([rows, col2]),
                               g * vz[pl.ds(o, 16)])
            return 0

        lax.fori_loop(0, _CH // 16, edge_step, 0)
        pltpu.sync_copy(dxyz, fpos_sh.at[idxs], add=True)
        pltpu.sync_copy(dxyz, fneg_sh.at[idxd], add=True)
    plsc.subcore_barrier()

    pltpu.sync_copy(fpos_sh.at[pl.ds(nbase, _NPT)], va)
    pltpu.sync_copy(fneg_sh.at[pl.ds(nbase, _NPT)], vb)

    def sub_step(m, _):
        c = m // (_NPT // 16)
        rows = iota + 16 * (m % (_NPT // 16))
        colv = jnp.full((16,), 1, jnp.int32) * c
        a = plsc.load_gather(va, ([rows, colv]))
        b = plsc.load_gather(vb, ([rows, colv]))
        plsc.store_scatter(va, ([rows, colv]), a - b)
        return 0

    lax.fori_loop(0, 3 * (_NPT // 16), sub_step, 0)
    pltpu.sync_copy(va, fout_hbm.at[pl.ds(nbase, _NPT)])

    @pl.when(wid == 0)
    def _():
        pltpu.sync_copy(es_sh, es_loc)
        acc = es_loc[0]
        for t in range(1, _NT):
            acc = acc + es_loc[t]
        tot = lax.broadcast(jnp.sum(acc, axis=0), (16,))
        es_loc[0] = tot
        pltpu.sync_copy(es_loc.at[0], esum_hbm)


def _sc_stage(d_e, c1_e, c0_e, rx, ry, rz, src, dst, coefs, zn, zf):
    mesh = plsc.VectorSubcoreMesh(core_axis_name="c", subcore_axis_name="s",
                                  num_cores=1, num_subcores=_NT)
    f32 = jnp.float32
    run = pl.kernel(
        _sc_body,
        out_type=[
            jax.ShapeDtypeStruct((_NPAD, 8), f32),
            jax.ShapeDtypeStruct((16,), f32),
        ],
        mesh=mesh,
        compiler_params=pltpu.CompilerParams(needs_layout_passes=False,
                                             use_tc_tiling_on_sc=False),
        scratch_types=[
            pltpu.VMEM_SHARED((_NPAD,), f32),
            pltpu.VMEM_SHARED((_NPAD,), f32),
            pltpu.VMEM_SHARED((_NPAD, 8), f32),
            pltpu.VMEM_SHARED((_NPAD, 8), f32),
            pltpu.VMEM_SHARED((_NT, 16), f32),
            pltpu.VMEM((_CH,), jnp.int32),
            pltpu.VMEM((_CH,), jnp.int32),
            pltpu.VMEM((_CH,), f32),
            pltpu.VMEM((_CH,), f32),
            pltpu.VMEM((_CH,), f32),
            pltpu.VMEM((_CH,), f32),
            pltpu.VMEM((_CH,), f32),
            pltpu.VMEM((_CH, 8), f32),
            pltpu.VMEM((_NPAD,), f32),
            pltpu.VMEM((_NPT,), f32),
            pltpu.VMEM((_NPT,), f32),
            pltpu.VMEM((10, 16), f32),
            pltpu.VMEM((_NPT, 8), f32),
            pltpu.VMEM((_NPT, 8), f32),
            pltpu.VMEM((_NT, 16), f32),
            pltpu.SemaphoreType.DMA,
        ],
    )
    return run(d_e, c1_e, c0_e, rx, ry, rz, src, dst, coefs, zn, zf)


_POWERS = np.concatenate([[0.5], 1.0 + np.arange(4)]).astype(np.float32)
_SF = np.concatenate(
    [[2.0], 1.0 / 10.0 ** np.cumsum(np.log10(1.0 + np.arange(4)))]
).astype(np.float32)


@jax.jit
def kernel(r, edge_index, phi_density, phi_pair, emb_weights):
    sp = jax.nn.softplus(phi_density)
    pp = phi_pair

    rr = np.arange(128)
    karr = 8 * (rr % 16) + rr // 16
    qarr = jnp.asarray(np.exp(-((rr // 16) ** 2) * _DELTA).astype(np.float32))
    cen = jnp.asarray((_DELTA * np.arange(128)).astype(np.float32))
    spk = sp[karr] * qarr
    ppk = pp[karr] * qarr
    ck = cen[karr]
    w4 = jnp.stack([spk, spk * ck, ppk, ppk * ck], axis=0)

    npad = _EPAD - N_EDGES
    rxp = jnp.pad(r[:, 0], (0, npad))
    ryp = jnp.pad(r[:, 1], (0, npad))
    rzp = jnp.pad(r[:, 2], (0, npad))

    d2, c12, c02, psum = _edge_stage(
        w4,
        rxp.reshape(_GRID, _ROWS, 128),
        ryp.reshape(_GRID, _ROWS, 128),
        rzp.reshape(_GRID, _ROWS, 128))

    cf = emb_weights * jnp.asarray(_SF)
    gfc = cf * jnp.asarray(_POWERS)
    coefs = jnp.broadcast_to(
        jnp.concatenate([cf, gfc])[:, None], (10, 16)).astype(jnp.float32)

    src_i = jnp.pad(edge_index[0].astype(jnp.int32), (0, npad))
    dst_i = jnp.pad(edge_index[1].astype(jnp.int32), (0, npad))
    zn = jnp.zeros((_NPT,), jnp.float32)
    zf = jnp.zeros((_NPT, 8), jnp.float32)

    fout, esum = _sc_stage(
        d2.reshape(_EPAD), c12.reshape(_EPAD), c02.reshape(_EPAD),
        rxp, ryp, rzp, src_i, dst_i, coefs, zn, zf)

    total_energy = (esum[0] + psum[0, 0]).reshape(1)
    forces = fout[:N_NODES, :3]
    return (total_energy, forces)


# speedup vs baseline: 14.1288x; 1.0010x over previous
import functools

import jax
import jax.numpy as jnp
import numpy as np
from jax import lax
from jax.experimental import pallas as pl
from jax.experimental.pallas import tpu as pltpu
from jax.experimental.pallas import tpu_sc as plsc

NBASIS = 128
CUTOFF = 6.0
N_NODES = 10000
N_EDGES = 320000

_DELTA = CUTOFF / (NBASIS - 1)
_GAMMA = 1.0 / _DELTA
_PI = float(np.pi)

_EPAD = 327680
_ROWS = 16
_GRID = _EPAD // (_ROWS * 128)


def _edge_body(w4_ref, rx_ref, ry_ref, rz_ref,
               d_ref, c1_ref, c0_ref, ps_ref):
    rx = rx_ref[0]
    ry = ry_ref[0]
    rz = rz_ref[0]
    l2 = rx * rx + ry * ry + rz * rz
    mask = l2 > 0.0
    l2s = jnp.maximum(l2, 1e-30)
    inv_l = lax.rsqrt(l2s)
    bl = l2s * inv_l
    th = (_PI / CUTOFF) * bl
    fc = 0.5 + 0.5 * jnp.cos(th)
    fcp = (-_PI / (2.0 * CUTOFF)) * jnp.sin(th)

    w4 = w4_ref[...]
    ccol = lax.broadcasted_iota(jnp.int32, (16, 1), 0).astype(jnp.float32) * (
        8.0 * _DELTA)

    sa_rows, sac_rows, sb_rows, sbc_rows = [], [], [], []
    _RPD = 16
    for g in range(_ROWS // _RPD):
        etiles = []
        for s in range(g * _RPD, (g + 1) * _RPD):
            blr = bl[s:s + 1, :]
            t0 = blr - ccol
            anchor = jnp.exp((-_GAMMA) * (t0 * t0))
            w = jnp.exp(2.0 * t0)
            rows = [anchor]
            x = anchor
            for _ in range(7):
                x = x * w
                rows.append(x)
            etiles.append(jnp.concatenate(rows, axis=0))
        ebig = jnp.concatenate(etiles, axis=1)
        r4 = lax.dot_general(w4, ebig, (((1,), (0,)), ((), ())),
                             precision=lax.Precision.HIGHEST,
                             preferred_element_type=jnp.float32)
        for q in range(_RPD):
            cs = q * 128
            sa_rows.append(r4[0:1, cs:cs + 128])
            sac_rows.append(r4[1:2, cs:cs + 128])
            sb_rows.append(r4[2:3, cs:cs + 128])
            sbc_rows.append(r4[3:4, cs:cs + 128])

    sa = jnp.concatenate(sa_rows, axis=0)
    sac = jnp.concatenate(sac_rows, axis=0)
    sb = jnp.concatenate(sb_rows, axis=0)
    sbc = jnp.concatenate(sbc_rows, axis=0)

    ap = (-2.0 * _GAMMA) * (bl * sa - sac)
    bp = (-2.0 * _GAMMA) * (bl * sb - sbc)
    zero = jnp.zeros_like(bl)
    d = jnp.where(mask, 0.5 * sa * fc, zero)
    c1 = jnp.where(mask, 0.5 * (ap * fc + sa * fcp) * inv_l, zero)
    p = jnp.where(mask, 0.5 * sb * fc * inv_l, zero)
    c0 = jnp.where(mask, ((bp * fc + sb * fcp) * 0.5 * inv_l - p * inv_l)
                   * inv_l, zero)

    d_ref[...] = d.reshape(1, _ROWS, 128)
    c1_ref[...] = c1.reshape(1, _ROWS, 128)
    c0_ref[...] = c0.reshape(1, _ROWS, 128)

    @pl.when(pl.program_id(0) == 0)
    def _():
        ps_ref[0, 0] = 0.0

    ps_ref[0, 0] += jnp.sum(p)


def _edge_stage(w4, rx2, ry2, rz2):
    blk = pl.BlockSpec((1, _ROWS, 128), lambda i: (i, 0, 0))
    return pl.pallas_call(
        _edge_body,
        grid=(_GRID,),
        in_specs=[
            pl.BlockSpec((4, 128), lambda i: (0, 0)),
            blk, blk, blk,
        ],
        out_specs=[
            blk, blk, blk,
            pl.BlockSpec((1, 1), lambda i: (0, 0), memory_space=pltpu.SMEM),
        ],
        out_shape=[
            jax.ShapeDtypeStruct((_GRID, _ROWS, 128), jnp.float32),
            jax.ShapeDtypeStruct((_GRID, _ROWS, 128), jnp.float32),
            jax.ShapeDtypeStruct((_GRID, _ROWS, 128), jnp.float32),
            jax.ShapeDtypeStruct((1, 1), jnp.float32),
        ],
    )(w4, rx2, ry2, rz2)


_NT = 16
_EPW = _EPAD // _NT
_NSUB = 4
_CH = _EPW // _NSUB
_NPAD = 10240
_NPT = _NPAD // _NT


def _rsqrt_newton(x):
    m = x
    sc = jnp.full((16,), 1.0, jnp.float32)
    for _ in range(16):
        big = m >= 4.0
        m = jnp.where(big, m * 0.25, m)
        sc = jnp.where(big, sc * 0.5, sc)
    for _ in range(32):
        small = m < 1.0
        m = jnp.where(small, m * 4.0, m)
        sc = jnp.where(small, sc * 2.0, sc)
    y = 1.30880086 + m * (-0.39516662 + m * 0.04939814)
    for _ in range(3):
        y = y * (1.5 - (0.5 * m) * y * y)
    return y * sc


def _sc_body(d_hbm, c1_hbm, c0_hbm, rx_hbm, ry_hbm, rz_hbm,
             src_hbm, dst_hbm, coef_hbm, zn_hbm, zf_hbm,
             fout_hbm, esum_hbm,
             rho_sh, fp_sh, fpos_sh, fneg_sh, es_sh,
             idxd, idxs, v0, v1, vx, vy, vz, dxyz,
             fp_loc, rho_loc, fp_part, coef_v, va, vb, es_loc, sem):
    wid = lax.axis_index("s")
    ebase = wid * _EPW
    nbase = wid * _NPT

    pltpu.sync_copy(zn_hbm, rho_loc)
    pltpu.sync_copy(rho_loc, rho_sh.at[pl.ds(nbase, _NPT)])
    pltpu.sync_copy(zf_hbm, va)
    pltpu.sync_copy(va, fpos_sh.at[pl.ds(nbase, _NPT)])
    pltpu.sync_copy(va, fneg_sh.at[pl.ds(nbase, _NPT)])
    for k in range(_CH // _NPT):
        pltpu.sync_copy(zf_hbm, dxyz.at[pl.ds(k * _NPT, _NPT)])
    pltpu.sync_copy(coef_hbm, coef_v)
    plsc.subcore_barrier()

    for s in range(_NSUB):
        off = ebase + s * _CH
        cps = [pltpu.async_copy(d_hbm.at[pl.ds(off, _CH)], v0, sem),
               pltpu.async_copy(dst_hbm.at[pl.ds(off, _CH)], idxd, sem)]
        for cp in cps:
            cp.wait()
        pltpu.sync_copy(v0, rho_sh.at[idxd], add=True)
    plsc.subcore_barrier()

    pltpu.sync_copy(rho_sh.at[pl.ds(nbase, _NPT)], rho_loc)
    cf = [coef_v[j] for j in range(5)]
    gf = [coef_v[5 + j] for j in range(5)]

    def node_step(i, facc):
        o = i * 16
        x = rho_loc[pl.ds(o, 16)]
        y = _rsqrt_newton(x)
        s = x * y
        fval = cf[0] * s + x * (cf[1] + x * (cf[2] + x * (cf[3] + x * cf[4])))
        fp = gf[0] * y + gf[1] + x * (gf[2] + x * (gf[3] + x * gf[4]))
        fp_part[pl.ds(o, 16)] = fp
        return facc + fval

    facc = lax.fori_loop(0, _NPT // 16, node_step, jnp.zeros((16,), jnp.float32))
    pltpu.sync_copy(fp_part, fp_sh.at[pl.ds(nbase, _NPT)])
    es_loc[0] = facc
    pltpu.sync_copy(es_loc.at[0], es_sh.at[wid])
    plsc.subcore_barrier()

    pltpu.sync_copy(fp_sh, fp_loc)
    iota = lax.iota(jnp.int32, 16)
    col0 = jnp.zeros((16,), jnp.int32)
    col1 = jnp.full((16,), 1, jnp.int32)
    col2 = jnp.full((16,), 2, jnp.int32)
    for s in range(_NSUB):
        off = ebase + s * _CH
        cps = [pltpu.async_copy(c1_hbm.at[pl.ds(off, _CH)], v0, sem),
               pltpu.async_copy(c0_hbm.at[pl.ds(off, _CH)], v1, sem),
               pltpu.async_copy(dst_hbm.at[pl.ds(off, _CH)], idxd, sem),
               pltpu.async_copy(src_hbm.at[pl.ds(off, _CH)], idxs, sem),
               pltpu.async_copy(rx_hbm.at[pl.ds(off, _CH)], vx, sem),
               pltpu.async_copy(ry_hbm.at[pl.ds(off, _CH)], vy, sem),
               pltpu.async_copy(rz_hbm.at[pl.ds(off, _CH)], vz, sem)]
        for cp in cps:
            cp.wait()

        def edge_step(j, _):
            o = j * 16
            dv = idxd[pl.ds(o, 16)]
            fpd = plsc.load_gather(fp_loc, ([dv]))
            g = fpd * v0[pl.ds(o, 16)] + v1[pl.ds(o, 16)]
            rows = iota + o
            plsc.store_scatter(dxyz, ([rows, col0]),
                               g * vx[pl.ds(o, 16)])
            plsc.store_scatter(dxyz, ([rows, col1]),
                               g * vy[pl.ds(o, 16)])
            plsc.store_scatter(dxyz, ([rows, col2]),
                               g * vz[pl.ds(o, 16)])
            return 0

        lax.fori_loop(0, _CH // 16, edge_step, 0)
        pltpu.sync_copy(dxyz, fpos_sh.at[idxs], add=True)
        pltpu.sync_copy(dxyz, fneg_sh.at[idxd], add=True)
    plsc.subcore_barrier()

    pltpu.sync_copy(fpos_sh.at[pl.ds(nbase, _NPT)], va)
    pltpu.sync_copy(fneg_sh.at[pl.ds(nbase, _NPT)], vb)

    def sub_step(m, _):
        c = m // (_NPT // 16)
        rows = iota + 16 * (m % (_NPT // 16))
        colv = jnp.full((16,), 1, jnp.int32) * c
        a = plsc.load_gather(va, ([rows, colv]))
        b = plsc.load_gather(vb, ([rows, colv]))
        plsc.store_scatter(va, ([rows, colv]), a - b)
        return 0

    lax.fori_loop(0, 3 * (_NPT // 16), sub_step, 0)
    pltpu.sync_copy(va, fout_hbm.at[pl.ds(nbase, _NPT)])

    @pl.when(wid == 0)
    def _():
        pltpu.sync_copy(es_sh, es_loc)
        acc = es_loc[0]
        for t in range(1, _NT):
            acc = acc + es_loc[t]
        tot = lax.broadcast(jnp.sum(acc, axis=0), (16,))
        es_loc[0] = tot
        pltpu.sync_copy(es_loc.at[0], esum_hbm)


def _sc_stage(d_e, c1_e, c0_e, rx, ry, rz, src, dst, coefs, zn, zf):
    mesh = plsc.VectorSubcoreMesh(core_axis_name="c", subcore_axis_name="s",
                                  num_cores=1, num_subcores=_NT)
    f32 = jnp.float32
    run = pl.kernel(
        _sc_body,
        out_type=[
            jax.ShapeDtypeStruct((_NPAD, 8), f32),
            jax.ShapeDtypeStruct((16,), f32),
        ],
        mesh=mesh,
        compiler_params=pltpu.CompilerParams(needs_layout_passes=False,
                                             use_tc_tiling_on_sc=False),
        scratch_types=[
            pltpu.VMEM_SHARED((_NPAD,), f32),
            pltpu.VMEM_SHARED((_NPAD,), f32),
            pltpu.VMEM_SHARED((_NPAD, 8), f32),
            pltpu.VMEM_SHARED((_NPAD, 8), f32),
            pltpu.VMEM_SHARED((_NT, 16), f32),
            pltpu.VMEM((_CH,), jnp.int32),
            pltpu.VMEM((_CH,), jnp.int32),
            pltpu.VMEM((_CH,), f32),
            pltpu.VMEM((_CH,), f32),
            pltpu.VMEM((_CH,), f32),
            pltpu.VMEM((_CH,), f32),
            pltpu.VMEM((_CH,), f32),
            pltpu.VMEM((_CH, 8), f32),
            pltpu.VMEM((_NPAD,), f32),
            pltpu.VMEM((_NPT,), f32),
            pltpu.VMEM((_NPT,), f32),
            pltpu.VMEM((10, 16), f32),
            pltpu.VMEM((_NPT, 8), f32),
            pltpu.VMEM((_NPT, 8), f32),
            pltpu.VMEM((_NT, 16), f32),
            pltpu.SemaphoreType.DMA,
        ],
    )
    return run(d_e, c1_e, c0_e, rx, ry, rz, src, dst, coefs, zn, zf)


_POWERS = np.concatenate([[0.5], 1.0 + np.arange(4)]).astype(np.float32)
_SF = np.concatenate(
    [[2.0], 1.0 / 10.0 ** np.cumsum(np.log10(1.0 + np.arange(4)))]
).astype(np.float32)


@jax.jit
def kernel(r, edge_index, phi_density, phi_pair, emb_weights):
    sp = jax.nn.softplus(phi_density)
    pp = phi_pair

    rr = np.arange(128)
    karr = 8 * (rr % 16) + rr // 16
    qarr = jnp.asarray(np.exp(-((rr // 16) ** 2) * _DELTA).astype(np.float32))
    cen = jnp.asarray((_DELTA * np.arange(128)).astype(np.float32))
    spk = sp[karr] * qarr
    ppk = pp[karr] * qarr
    ck = cen[karr]
    w4 = jnp.stack([spk, spk * ck, ppk, ppk * ck], axis=0)

    npad = _EPAD - N_EDGES
    rxp = jnp.pad(r[:, 0], (0, npad))
    ryp = jnp.pad(r[:, 1], (0, npad))
    rzp = jnp.pad(r[:, 2], (0, npad))

    d2, c12, c02, psum = _edge_stage(
        w4,
        rxp.reshape(_GRID, _ROWS, 128),
        ryp.reshape(_GRID, _ROWS, 128),
        rzp.reshape(_GRID, _ROWS, 128))

    cf = emb_weights * jnp.asarray(_SF)
    gfc = cf * jnp.asarray(_POWERS)
    coefs = jnp.broadcast_to(
        jnp.concatenate([cf, gfc])[:, None], (10, 16)).astype(jnp.float32)

    src_i = jnp.pad(edge_index[0].astype(jnp.int32), (0, npad))
    dst_i = jnp.pad(edge_index[1].astype(jnp.int32), (0, npad))
    zn = jnp.zeros((_NPT,), jnp.float32)
    zf = jnp.zeros((_NPT, 8), jnp.float32)

    fout, esum = _sc_stage(
        d2.reshape(_EPAD), c12.reshape(_EPAD), c02.reshape(_EPAD),
        rxp, ryp, rzp, src_i, dst_i, coefs, zn, zf)

    total_energy = (esum[0] + psum[0, 0]).reshape(1)
    forces = fout[:N_NODES, :3]
    return (total_energy, forces)
